# baseline (speedup 1.0000x reference)
import functools

import jax
import jax.numpy as jnp
from jax import lax
from jax.experimental import pallas as pl
from jax.experimental.pallas import tpu as pltpu
from jax.experimental.pallas import tpu_sc as plsc

_N = 100000
_E = 1600000
_NP = 100352
_NC, _NS, _L = 2, 16, 16
_NW = _NC * _NS
_K = 400
_NCH = 128
_EP = _NW * _K * _NCH
_EPW = _EP // _NW
_NPW = _NP // _NS

_mesh = plsc.VectorSubcoreMesh(
    core_axis_name="c", subcore_axis_name="s", num_cores=_NC, num_subcores=_NS)
_sc_params = pltpu.CompilerParams(
    needs_layout_passes=False, use_tc_tiling_on_sc=False)


_RING = 4


@functools.partial(
    pl.kernel,
    out_type=jax.ShapeDtypeStruct((_NC, _NP, 16), jnp.float32),
    mesh=_mesh,
    scratch_types=[
        pltpu.VMEM((_RING, 3, _K), jnp.int32),
        pltpu.VMEM((_RING, _K, 16), jnp.float32),
        pltpu.VMEM_SHARED((_NP, 16), jnp.float32),
    ] + [pltpu.SemaphoreType.DMA] * (3 * _RING),
    compiler_params=_sc_params,
)
def _agg16(hs_hbm, pk_hbm, z16_hbm, out_hbm, ib, rb, acc_sh, *sems):
    cid = lax.axis_index("c")
    sid = lax.axis_index("s")
    gsem = sems[0:_RING]
    ssem = sems[_RING:2 * _RING]
    isem = sems[2 * _RING:3 * _RING]
    pltpu.sync_copy(z16_hbm.at[pl.ds(sid * _NPW, _NPW)],
                    acc_sh.at[pl.ds(sid * _NPW, _NPW)])
    plsc.subcore_barrier()
    base = (cid * _NS + sid) * _NCH

    pltpu.sync_copy(pk_hbm.at[base], ib.at[0])
    pltpu.async_copy(hs_hbm.at[ib.at[0, 0]], rb.at[0], gsem[0])
    pltpu.async_copy(pk_hbm.at[base + 1], ib.at[1], isem[1])

    def body(i, carry):
        for b in range(_RING):
            j = _RING * i + b
            b1 = (b + 1) % _RING
            b2 = (b + 2) % _RING

            def prep_idx():
                @pl.when(j >= 2)
                def _():
                    pltpu.make_async_copy(
                        rb.at[b2], acc_sh.at[ib.at[b2, 1]], ssem[b2]).wait()
                pltpu.async_copy(pk_hbm.at[base + j + 2], ib.at[b2], isem[b2])

            pl.when(j + 2 < _NCH)(prep_idx)

            def issue_gather():
                pltpu.make_async_copy(
                    pk_hbm.at[base + j + 1], ib.at[b1], isem[b1]).wait()
                pltpu.async_copy(hs_hbm.at[ib.at[b1, 0]], rb.at[b1], gsem[b1])

            pl.when(j + 1 < _NCH)(issue_gather)
            pltpu.make_async_copy(
                hs_hbm.at[ib.at[b, 0]], rb.at[b], gsem[b]).wait()

            def scale(g, c):
                w16 = plsc.bitcast(ib[b, 2, pl.ds(g * _L, _L)], jnp.float32)
                for u in range(_L):
                    k = g * _L + u
                    rb[b, k, :] = rb[b, k, :] * w16[u]
                return c
            lax.fori_loop(0, _K // _L, scale, 0, unroll=1)
            pltpu.async_copy(rb.at[b], acc_sh.at[ib.at[b, 1]], ssem[b],
                             add=True)
        return carry

    lax.fori_loop(0, _NCH // _RING, body, 0)
    for b in range(_RING):
        pltpu.make_async_copy(rb.at[b], acc_sh.at[ib.at[b, 1]], ssem[b]).wait()
    plsc.subcore_barrier()
    pltpu.sync_copy(acc_sh.at[pl.ds(sid * _NPW, _NPW)],
                    out_hbm.at[cid, pl.ds(sid * _NPW, _NPW)])


@functools.partial(
    pl.kernel,
    out_type=jax.ShapeDtypeStruct((_NC, _NP), jnp.float32),
    mesh=_mesh,
    scratch_types=[
        pltpu.VMEM((_NP,), jnp.float32),
        pltpu.VMEM((_RING, 3, _K), jnp.int32),
        pltpu.VMEM((_RING, _K), jnp.float32),
        pltpu.VMEM_SHARED((_NP,), jnp.float32),
    ] + [pltpu.SemaphoreType.DMA] * (2 * _RING),
    compiler_params=_sc_params,
)
def _agg1(vs_hbm, pk_hbm, z1_hbm, out_hbm, vs_v, ib, vals, acc_sh, *sems):
    cid = lax.axis_index("c")
    sid = lax.axis_index("s")
    ssem = sems[0:_RING]
    isem = sems[_RING:2 * _RING]
    pltpu.sync_copy(vs_hbm, vs_v)
    pltpu.sync_copy(z1_hbm.at[pl.ds(sid * _NPW, _NPW)],
                    acc_sh.at[pl.ds(sid * _NPW, _NPW)])
    plsc.subcore_barrier()
    base = (cid * _NS + sid) * _NCH

    pltpu.sync_copy(pk_hbm.at[base], ib.at[0])
    pltpu.async_copy(pk_hbm.at[base + 1], ib.at[1], isem[1])

    def body(i, carry):
        for b in range(_RING):
            j = _RING * i + b
            b2 = (b + 2) % _RING

            def prep_idx():
                @pl.when(j >= 2)
                def _():
                    pltpu.make_async_copy(
                        vals.at[b2], acc_sh.at[ib.at[b2, 1]], ssem[b2]).wait()
                pltpu.async_copy(pk_hbm.at[base + j + 2], ib.at[b2], isem[b2])

            pl.when(j + 2 < _NCH)(prep_idx)

            @pl.when(j >= 1)
            def _():
                pltpu.make_async_copy(
                    pk_hbm.at[base + j], ib.at[b], isem[b]).wait()

            def grp(g, c):
                idx = ib[b, 0, pl.ds(g * _L, _L)]
                w16 = plsc.bitcast(ib[b, 2, pl.ds(g * _L, _L)], jnp.float32)
                vals[b, pl.ds(g * _L, _L)] = plsc.load_gather(vs_v, [idx]) * w16
                return c
            lax.fori_loop(0, _K // _L, grp, 0, unroll=1)
            pltpu.async_copy(vals.at[b], acc_sh.at[ib.at[b, 1]], ssem[b],
                             add=True)
        return carry

    lax.fori_loop(0, _NCH // _RING, body, 0)
    for b in range(_RING):
        pltpu.make_async_copy(vals.at[b], acc_sh.at[ib.at[b, 1]],
                              ssem[b]).wait()
    plsc.subcore_barrier()
    pltpu.sync_copy(acc_sh.at[pl.ds(sid * _NPW, _NPW)],
                    out_hbm.at[cid, pl.ds(sid * _NPW, _NPW)])


_BR = 2048
_GRID = _NP // _BR


def _lrelu(a):
    return jnp.where(a > 0, a, 0.01 * a)


def _vspec():
    return pl.BlockSpec((_BR,), lambda i: (i,))


def _mspec(c):
    return pl.BlockSpec((_BR, c), lambda i: (i, 0))


def _pspec16():
    return pl.BlockSpec((2, _BR, 16), lambda i: (0, i, 0))


def _pspec1():
    return pl.BlockSpec((2, _BR), lambda i: (0, i))


def _wspec(shape):
    nd = len(shape)
    return pl.BlockSpec(shape, lambda i: (0,) * nd)


def _prep_body(degp_ref, x_ref, dinv_ref, selfn_ref, xs_ref):
    deg = degp_ref[0] + degp_ref[1] + 1.0
    dinv = lax.rsqrt(deg)
    dinv_ref[...] = dinv
    selfn_ref[...] = dinv * dinv
    xs_ref[:, :13] = x_ref[...] * dinv[:, None]
    xs_ref[:, 13:] = jnp.zeros((_BR, 3), jnp.float32)


_prep = pl.pallas_call(
    _prep_body,
    grid=(_GRID,),
    in_specs=[_pspec1(), _mspec(13)],
    out_specs=(_vspec(), _vspec(), _mspec(16)),
    out_shape=(jax.ShapeDtypeStruct((_NP,), jnp.float32),
               jax.ShapeDtypeStruct((_NP,), jnp.float32),
               jax.ShapeDtypeStruct((_NP, 16), jnp.float32)),
)


def _prep2_body(pa_ref, x_ref, dinv_ref, selfn_ref, w0_ref, b0_ref,
                aggx_ref, stm_ref, ytm_ref, h_ref, hs_ref):
    s = pa_ref[0] + pa_ref[1]
    x = x_ref[...]
    dinv = dinv_ref[...][:, None]
    selfn = selfn_ref[...][:, None]
    aggx = dinv * (s[:, 1:13] - s[:, 0:12]) + selfn * (x[:, 1:] - x[:, :-1])
    aggx_ref[...] = aggx
    stm_ref[...] = s[:, 12]
    ytm_ref[...] = x[:, 12]
    h = _lrelu(jnp.dot(aggx, w0_ref[...], preferred_element_type=jnp.float32)
               + b0_ref[...])
    h_ref[...] = h
    hs_ref[...] = h * dinv


_prep2 = pl.pallas_call(
    _prep2_body,
    grid=(_GRID,),
    in_specs=[_pspec16(), _mspec(13), _vspec(), _vspec(),
              _wspec((12, 16)), _wspec((16,))],
    out_specs=(_mspec(12), _vspec(), _vspec(), _mspec(16), _mspec(16)),
    out_shape=(jax.ShapeDtypeStruct((_NP, 12), jnp.float32),
               jax.ShapeDtypeStruct((_NP,), jnp.float32),
               jax.ShapeDtypeStruct((_NP,), jnp.float32),
               jax.ShapeDtypeStruct((_NP, 16), jnp.float32),
               jax.ShapeDtypeStruct((_NP, 16), jnp.float32)),
)


def _roll_body(aggx_ref, py_ref, stm_ref, ytm_ref, yp_ref, dinv_ref,
               selfn_ref, w0_ref, b0_ref,
               aggx2_ref, stm2_ref, h_ref, hs_ref):
    dinv = dinv_ref[...][:, None]
    selfn = selfn_ref[...][:, None]
    pysum = py_ref[0] + py_ref[1]
    newcol = (dinv[:, 0] * (pysum - stm_ref[...])
              + selfn[:, 0] * (yp_ref[...] - ytm_ref[...]))
    aggx = jnp.concatenate([aggx_ref[...][:, 1:], newcol[:, None]], axis=1)
    aggx2_ref[...] = aggx
    stm2_ref[...] = pysum
    h = _lrelu(jnp.dot(aggx, w0_ref[...], preferred_element_type=jnp.float32)
               + b0_ref[...])
    h_ref[...] = h
    hs_ref[...] = h * dinv


_stage_roll = pl.pallas_call(
    _roll_body,
    grid=(_GRID,),
    in_specs=[_mspec(12), _pspec1(), _vspec(), _vspec(), _vspec(),
              _vspec(), _vspec(), _wspec((12, 16)), _wspec((16,))],
    out_specs=(_mspec(12), _vspec(), _mspec(16), _mspec(16)),
    out_shape=(jax.ShapeDtypeStruct((_NP, 12), jnp.float32),
               jax.ShapeDtypeStruct((_NP,), jnp.float32),
               jax.ShapeDtypeStruct((_NP, 16), jnp.float32),
               jax.ShapeDtypeStruct((_NP, 16), jnp.float32)),
)


def _h1_body(p_ref, dinv_ref, selfn_ref, h_ref, w1_ref, b1_ref,
             h1_ref, h1s_ref, aggh_ref):
    dinv = dinv_ref[...][:, None]
    selfn = selfn_ref[...][:, None]
    aggh = dinv * (p_ref[0] + p_ref[1]) + selfn * h_ref[...]
    h1 = _lrelu(jnp.dot(aggh, w1_ref[...], preferred_element_type=jnp.float32)
                + b1_ref[...])
    h1_ref[...] = h1
    h1s_ref[...] = h1 * dinv
    aggh_ref[...] = aggh


_stage_h1 = pl.pallas_call(
    _h1_body,
    grid=(_GRID,),
    in_specs=[_pspec16(), _vspec(), _vspec(), _mspec(16),
              _wspec((16, 16)), _wspec((16,))],
    out_specs=(_mspec(16), _mspec(16), _mspec(16)),
    out_shape=(jax.ShapeDtypeStruct((_NP, 16), jnp.float32),
               jax.ShapeDtypeStruct((_NP, 16), jnp.float32),
               jax.ShapeDtypeStruct((_NP, 16), jnp.float32)),
)


def _z_body(p_ref, dinv_ref, selfn_ref, h1_ref, aggx_ref, aggh_ref,
            w2_ref, b2_ref, w3_ref, z_ref, zs_ref):
    dinv = dinv_ref[...][:, None]
    selfn = selfn_ref[...][:, None]
    aggh1 = dinv * (p_ref[0] + p_ref[1]) + selfn * h1_ref[...]
    aggtemp = jnp.concatenate([aggx_ref[...], aggh_ref[...], aggh1], axis=1)
    xt = _lrelu(jnp.dot(aggtemp, w2_ref[...],
                        preferred_element_type=jnp.float32) + b2_ref[...])
    z = jnp.dot(xt, w3_ref[...], preferred_element_type=jnp.float32)[:, 0]
    z_ref[...] = z
    zs_ref[...] = z * dinv[:, 0]


_stage_z = pl.pallas_call(
    _z_body,
    grid=(_GRID,),
    in_specs=[_pspec16(), _vspec(), _vspec(), _mspec(16), _mspec(12),
              _mspec(16), _wspec((44, 44)), _wspec((44,)), _wspec((44, 1))],
    out_specs=(_vspec(), _vspec()),
    out_shape=(jax.ShapeDtypeStruct((_NP,), jnp.float32),
               jax.ShapeDtypeStruct((_NP,), jnp.float32)),
)


def _y_body(p_ref, dinv_ref, selfn_ref, z_ref, ytm_ref, b3_ref,
            yp_ref, ys_ref):
    dinv = dinv_ref[...]
    yp = (ytm_ref[...] + dinv * (p_ref[0] + p_ref[1])
          + selfn_ref[...] * z_ref[...] + b3_ref[0])
    yp_ref[...] = yp
    ys_ref[...] = yp * dinv


_stage_y = pl.pallas_call(
    _y_body,
    grid=(_GRID,),
    in_specs=[_pspec1(), _vspec(), _vspec(), _vspec(), _vspec(),
              pl.BlockSpec(memory_space=pltpu.SMEM)],
    out_specs=(_vspec(), _vspec()),
    out_shape=(jax.ShapeDtypeStruct((_NP,), jnp.float32),
               jax.ShapeDtypeStruct((_NP,), jnp.float32)),
)


def kernel(x, y, edge_index, edge_attr, W0, b0, W1, b1, W2, b2, W3, b3):
    t_fut = y.shape[1]
    n = x.shape[0]
    xp = jnp.pad(x, ((0, _NP - n), (0, 0)))
    row = jnp.pad(edge_index[0], (0, _EP - _E)).reshape(_NW, _NCH, _K)
    col = jnp.pad(edge_index[1], (0, _EP - _E)).reshape(_NW, _NCH, _K)
    wi = lax.bitcast_convert_type(
        jnp.pad(edge_attr, (0, _EP - _E)), jnp.int32).reshape(_NW, _NCH, _K)
    pk = jnp.stack([row, col, wi], axis=2).reshape(_NW * _NCH, 3, _K)
    z16 = jnp.zeros((_NP, 16), jnp.float32)
    z1 = jnp.zeros((_NP,), jnp.float32)
    ones = jnp.ones((_NP,), jnp.float32)

    degp = _agg1(ones, pk, z1)
    dinv, selfn, xs = _prep(degp, xp)
    pa = _agg16(xs, pk, z16)
    aggx, stm, ytm, h, hs = _prep2(pa, xp, dinv, selfn, W0, b0)
    preds = []
    yp = py = None
    for t in range(t_fut):
        if t > 0:
            aggx, stm, h, hs = _stage_roll(aggx, py, stm, ytm, yp,
                                           dinv, selfn, W0, b0)
            ytm = yp
        ph = _agg16(hs, pk, z16)
        h1, h1s, aggh = _stage_h1(ph, dinv, selfn, h, W1, b1)
        ph1 = _agg16(h1s, pk, z16)
        zv, zs = _stage_z(ph1, dinv, selfn, h1, aggx, aggh, W2, b2, W3)
        pz = _agg1(zs, pk, z1)
        yp, ys = _stage_y(pz, dinv, selfn, zv, ytm, b3)
        preds.append(yp)
        if t + 1 < t_fut:
            py = _agg1(ys, pk, z1)
    return jnp.stack(preds, axis=1)[:n]


# speedup vs baseline: 1.2365x; 1.2365x over previous
import functools

import jax
import jax.numpy as jnp
from jax import lax
from jax.experimental import pallas as pl
from jax.experimental.pallas import tpu as pltpu
from jax.experimental.pallas import tpu_sc as plsc

_N = 100000
_E = 1600000
_NP = 100352
_NC, _NS, _L = 2, 16, 16
_NW = _NC * _NS
_K = 384
_NCH = 132
_EP = _NW * _K * _NCH
_EPW = _EP // _NW
_NPW = _NP // _NS

_mesh = plsc.VectorSubcoreMesh(
    core_axis_name="c", subcore_axis_name="s", num_cores=_NC, num_subcores=_NS)
_sc_params = pltpu.CompilerParams(
    needs_layout_passes=False, use_tc_tiling_on_sc=False)


_RING = 4


@functools.partial(
    pl.kernel,
    out_type=jax.ShapeDtypeStruct((_NC, _NP, 16), jnp.float32),
    mesh=_mesh,
    scratch_types=[
        pltpu.VMEM((_RING, 3, _K), jnp.int32),
        pltpu.VMEM((_RING, _K, 16), jnp.float32),
        pltpu.VMEM_SHARED((_NP, 16), jnp.float32),
    ] + [pltpu.SemaphoreType.DMA] * (3 * _RING),
    compiler_params=_sc_params,
)
def _agg16(hs_hbm, pk_hbm, z16_hbm, out_hbm, ib, rb, acc_sh, *sems):
    cid = lax.axis_index("c")
    sid = lax.axis_index("s")
    gsem = sems[0:_RING]
    ssem = sems[_RING:2 * _RING]
    isem = sems[2 * _RING:3 * _RING]
    pltpu.sync_copy(z16_hbm.at[pl.ds(sid * _NPW, _NPW)],
                    acc_sh.at[pl.ds(sid * _NPW, _NPW)])
    plsc.subcore_barrier()
    base = (cid * _NS + sid) * _NCH

    pltpu.sync_copy(pk_hbm.at[base], ib.at[0])
    pltpu.async_copy(hs_hbm.at[ib.at[0, 0]], rb.at[0], gsem[0])
    pltpu.async_copy(pk_hbm.at[base + 1], ib.at[1], isem[1])

    def body(i, carry):
        for b in range(_RING):
            j = _RING * i + b
            b1 = (b + 1) % _RING
            b2 = (b + 2) % _RING

            def prep_idx():
                @pl.when(j >= 2)
                def _():
                    pltpu.make_async_copy(
                        rb.at[b2], acc_sh.at[ib.at[b2, 1]], ssem[b2]).wait()
                pltpu.async_copy(pk_hbm.at[base + j + 2], ib.at[b2], isem[b2])

            pl.when(j + 2 < _NCH)(prep_idx)

            def issue_gather():
                pltpu.make_async_copy(
                    pk_hbm.at[base + j + 1], ib.at[b1], isem[b1]).wait()
                pltpu.async_copy(hs_hbm.at[ib.at[b1, 0]], rb.at[b1], gsem[b1])

            pl.when(j + 1 < _NCH)(issue_gather)
            pltpu.make_async_copy(
                hs_hbm.at[ib.at[b, 0]], rb.at[b], gsem[b]).wait()

            def scale(g, c):
                w16 = plsc.bitcast(ib[b, 2, pl.ds(g * _L, _L)], jnp.float32)
                for u in range(_L):
                    k = g * _L + u
                    rb[b, k, :] = rb[b, k, :] * w16[u]
                return c
            lax.fori_loop(0, _K // _L, scale, 0, unroll=1)
            pltpu.async_copy(rb.at[b], acc_sh.at[ib.at[b, 1]], ssem[b],
                             add=True)
        return carry

    lax.fori_loop(0, _NCH // _RING, body, 0)
    for b in range(_RING):
        pltpu.make_async_copy(rb.at[b], acc_sh.at[ib.at[b, 1]], ssem[b]).wait()
    plsc.subcore_barrier()
    pltpu.sync_copy(acc_sh.at[pl.ds(sid * _NPW, _NPW)],
                    out_hbm.at[cid, pl.ds(sid * _NPW, _NPW)])


@functools.partial(
    pl.kernel,
    out_type=jax.ShapeDtypeStruct((_NC, _NP), jnp.float32),
    mesh=_mesh,
    scratch_types=[
        pltpu.VMEM((_NP,), jnp.float32),
        pltpu.VMEM((_RING, 3, _K), jnp.int32),
        pltpu.VMEM((_RING, _K), jnp.float32),
        pltpu.VMEM_SHARED((_NP,), jnp.float32),
    ] + [pltpu.SemaphoreType.DMA] * (2 * _RING),
    compiler_params=_sc_params,
)
def _agg1(vs_hbm, pk_hbm, z1_hbm, out_hbm, vs_v, ib, vals, acc_sh, *sems):
    cid = lax.axis_index("c")
    sid = lax.axis_index("s")
    ssem = sems[0:_RING]
    isem = sems[_RING:2 * _RING]
    pltpu.sync_copy(vs_hbm, vs_v)
    pltpu.sync_copy(z1_hbm.at[pl.ds(sid * _NPW, _NPW)],
                    acc_sh.at[pl.ds(sid * _NPW, _NPW)])
    plsc.subcore_barrier()
    base = (cid * _NS + sid) * _NCH

    pltpu.sync_copy(pk_hbm.at[base], ib.at[0])
    pltpu.async_copy(pk_hbm.at[base + 1], ib.at[1], isem[1])

    def body(i, carry):
        for b in range(_RING):
            j = _RING * i + b
            b2 = (b + 2) % _RING

            def prep_idx():
                @pl.when(j >= 2)
                def _():
                    pltpu.make_async_copy(
                        vals.at[b2], acc_sh.at[ib.at[b2, 1]], ssem[b2]).wait()
                pltpu.async_copy(pk_hbm.at[base + j + 2], ib.at[b2], isem[b2])

            pl.when(j + 2 < _NCH)(prep_idx)

            @pl.when(j >= 1)
            def _():
                pltpu.make_async_copy(
                    pk_hbm.at[base + j], ib.at[b], isem[b]).wait()

            def grp(g, c):
                idx = ib[b, 0, pl.ds(g * _L, _L)]
                w16 = plsc.bitcast(ib[b, 2, pl.ds(g * _L, _L)], jnp.float32)
                vals[b, pl.ds(g * _L, _L)] = plsc.load_gather(vs_v, [idx]) * w16
                return c
            lax.fori_loop(0, _K // _L, grp, 0, unroll=1)
            pltpu.async_copy(vals.at[b], acc_sh.at[ib.at[b, 1]], ssem[b],
                             add=True)
        return carry

    lax.fori_loop(0, _NCH // _RING, body, 0)
    for b in range(_RING):
        pltpu.make_async_copy(vals.at[b], acc_sh.at[ib.at[b, 1]],
                              ssem[b]).wait()
    plsc.subcore_barrier()
    pltpu.sync_copy(acc_sh.at[pl.ds(sid * _NPW, _NPW)],
                    out_hbm.at[cid, pl.ds(sid * _NPW, _NPW)])


_BR = 2048
_GRID = _NP // _BR


def _lrelu(a):
    return jnp.where(a > 0, a, 0.01 * a)


def _vspec():
    return pl.BlockSpec((_BR,), lambda i: (i,))


def _mspec(c):
    return pl.BlockSpec((_BR, c), lambda i: (i, 0))


def _pspec16():
    return pl.BlockSpec((2, _BR, 16), lambda i: (0, i, 0))


def _pspec1():
    return pl.BlockSpec((2, _BR), lambda i: (0, i))


def _wspec(shape):
    nd = len(shape)
    return pl.BlockSpec(shape, lambda i: (0,) * nd)


def _prep_body(degp_ref, x_ref, dinv_ref, selfn_ref, xs_ref):
    deg = degp_ref[0] + degp_ref[1] + 1.0
    dinv = lax.rsqrt(deg)
    dinv_ref[...] = dinv
    selfn_ref[...] = dinv * dinv
    xs_ref[:, :13] = x_ref[...] * dinv[:, None]
    xs_ref[:, 13:] = jnp.zeros((_BR, 3), jnp.float32)


_prep = pl.pallas_call(
    _prep_body,
    grid=(_GRID,),
    in_specs=[_pspec1(), _mspec(13)],
    out_specs=(_vspec(), _vspec(), _mspec(16)),
    out_shape=(jax.ShapeDtypeStruct((_NP,), jnp.float32),
               jax.ShapeDtypeStruct((_NP,), jnp.float32),
               jax.ShapeDtypeStruct((_NP, 16), jnp.float32)),
)


def _prep2_body(pa_ref, x_ref, dinv_ref, selfn_ref, w0_ref, b0_ref,
                aggx_ref, stm_ref, ytm_ref, h_ref, hs_ref):
    s = pa_ref[0] + pa_ref[1]
    x = x_ref[...]
    dinv = dinv_ref[...][:, None]
    selfn = selfn_ref[...][:, None]
    aggx = dinv * (s[:, 1:13] - s[:, 0:12]) + selfn * (x[:, 1:] - x[:, :-1])
    aggx_ref[...] = aggx
    stm_ref[...] = s[:, 12]
    ytm_ref[...] = x[:, 12]
    h = _lrelu(jnp.dot(aggx, w0_ref[...], preferred_element_type=jnp.float32)
               + b0_ref[...])
    h_ref[...] = h
    hs_ref[...] = h * dinv


_prep2 = pl.pallas_call(
    _prep2_body,
    grid=(_GRID,),
    in_specs=[_pspec16(), _mspec(13), _vspec(), _vspec(),
              _wspec((12, 16)), _wspec((16,))],
    out_specs=(_mspec(12), _vspec(), _vspec(), _mspec(16), _mspec(16)),
    out_shape=(jax.ShapeDtypeStruct((_NP, 12), jnp.float32),
               jax.ShapeDtypeStruct((_NP,), jnp.float32),
               jax.ShapeDtypeStruct((_NP,), jnp.float32),
               jax.ShapeDtypeStruct((_NP, 16), jnp.float32),
               jax.ShapeDtypeStruct((_NP, 16), jnp.float32)),
)


def _roll_body(aggx_ref, py_ref, stm_ref, ytm_ref, yp_ref, dinv_ref,
               selfn_ref, w0_ref, b0_ref,
               aggx2_ref, stm2_ref, h_ref, hs_ref):
    dinv = dinv_ref[...][:, None]
    selfn = selfn_ref[...][:, None]
    pysum = py_ref[0] + py_ref[1]
    newcol = (dinv[:, 0] * (pysum - stm_ref[...])
              + selfn[:, 0] * (yp_ref[...] - ytm_ref[...]))
    aggx = jnp.concatenate([aggx_ref[...][:, 1:], newcol[:, None]], axis=1)
    aggx2_ref[...] = aggx
    stm2_ref[...] = pysum
    h = _lrelu(jnp.dot(aggx, w0_ref[...], preferred_element_type=jnp.float32)
               + b0_ref[...])
    h_ref[...] = h
    hs_ref[...] = h * dinv


_stage_roll = pl.pallas_call(
    _roll_body,
    grid=(_GRID,),
    in_specs=[_mspec(12), _pspec1(), _vspec(), _vspec(), _vspec(),
              _vspec(), _vspec(), _wspec((12, 16)), _wspec((16,))],
    out_specs=(_mspec(12), _vspec(), _mspec(16), _mspec(16)),
    out_shape=(jax.ShapeDtypeStruct((_NP, 12), jnp.float32),
               jax.ShapeDtypeStruct((_NP,), jnp.float32),
               jax.ShapeDtypeStruct((_NP, 16), jnp.float32),
               jax.ShapeDtypeStruct((_NP, 16), jnp.float32)),
)


def _h1_body(p_ref, dinv_ref, selfn_ref, h_ref, w1_ref, b1_ref,
             h1_ref, h1s_ref, aggh_ref):
    dinv = dinv_ref[...][:, None]
    selfn = selfn_ref[...][:, None]
    aggh = dinv * (p_ref[0] + p_ref[1]) + selfn * h_ref[...]
    h1 = _lrelu(jnp.dot(aggh, w1_ref[...], preferred_element_type=jnp.float32)
                + b1_ref[...])
    h1_ref[...] = h1
    h1s_ref[...] = h1 * dinv
    aggh_ref[...] = aggh


_stage_h1 = pl.pallas_call(
    _h1_body,
    grid=(_GRID,),
    in_specs=[_pspec16(), _vspec(), _vspec(), _mspec(16),
              _wspec((16, 16)), _wspec((16,))],
    out_specs=(_mspec(16), _mspec(16), _mspec(16)),
    out_shape=(jax.ShapeDtypeStruct((_NP, 16), jnp.float32),
               jax.ShapeDtypeStruct((_NP, 16), jnp.float32),
               jax.ShapeDtypeStruct((_NP, 16), jnp.float32)),
)


def _z_body(p_ref, dinv_ref, selfn_ref, h1_ref, aggx_ref, aggh_ref,
            w2_ref, b2_ref, w3_ref, z_ref, zs_ref):
    dinv = dinv_ref[...][:, None]
    selfn = selfn_ref[...][:, None]
    aggh1 = dinv * (p_ref[0] + p_ref[1]) + selfn * h1_ref[...]
    aggtemp = jnp.concatenate([aggx_ref[...], aggh_ref[...], aggh1], axis=1)
    xt = _lrelu(jnp.dot(aggtemp, w2_ref[...],
                        preferred_element_type=jnp.float32) + b2_ref[...])
    z = jnp.dot(xt, w3_ref[...], preferred_element_type=jnp.float32)[:, 0]
    z_ref[...] = z
    zs_ref[...] = z * dinv[:, 0]


_stage_z = pl.pallas_call(
    _z_body,
    grid=(_GRID,),
    in_specs=[_pspec16(), _vspec(), _vspec(), _mspec(16), _mspec(12),
              _mspec(16), _wspec((44, 44)), _wspec((44,)), _wspec((44, 1))],
    out_specs=(_vspec(), _vspec()),
    out_shape=(jax.ShapeDtypeStruct((_NP,), jnp.float32),
               jax.ShapeDtypeStruct((_NP,), jnp.float32)),
)


def _y_body(p_ref, dinv_ref, selfn_ref, z_ref, ytm_ref, b3_ref,
            yp_ref, ys_ref):
    dinv = dinv_ref[...]
    yp = (ytm_ref[...] + dinv * (p_ref[0] + p_ref[1])
          + selfn_ref[...] * z_ref[...] + b3_ref[0])
    yp_ref[...] = yp
    ys_ref[...] = yp * dinv


_stage_y = pl.pallas_call(
    _y_body,
    grid=(_GRID,),
    in_specs=[_pspec1(), _vspec(), _vspec(), _vspec(), _vspec(),
              pl.BlockSpec(memory_space=pltpu.SMEM)],
    out_specs=(_vspec(), _vspec()),
    out_shape=(jax.ShapeDtypeStruct((_NP,), jnp.float32),
               jax.ShapeDtypeStruct((_NP,), jnp.float32)),
)


def kernel(x, y, edge_index, edge_attr, W0, b0, W1, b1, W2, b2, W3, b3):
    t_fut = y.shape[1]
    n = x.shape[0]
    xp = jnp.pad(x, ((0, _NP - n), (0, 0)))
    row = jnp.pad(edge_index[0], (0, _EP - _E)).reshape(_NW, _NCH, _K)
    col = jnp.pad(edge_index[1], (0, _EP - _E)).reshape(_NW, _NCH, _K)
    wi = lax.bitcast_convert_type(
        jnp.pad(edge_attr, (0, _EP - _E)), jnp.int32).reshape(_NW, _NCH, _K)
    pk = jnp.stack([row, col, wi], axis=2).reshape(_NW * _NCH, 3, _K)
    z16 = jnp.zeros((_NP, 16), jnp.float32)
    z1 = jnp.zeros((_NP,), jnp.float32)
    ones = jnp.ones((_NP,), jnp.float32)

    degp = _agg1(ones, pk, z1)
    dinv, selfn, xs = _prep(degp, xp)
    pa = _agg16(xs, pk, z16)
    aggx, stm, ytm, h, hs = _prep2(pa, xp, dinv, selfn, W0, b0)
    preds = []
    yp = py = None
    for t in range(t_fut):
        if t > 0:
            aggx, stm, h, hs = _stage_roll(aggx, py, stm, ytm, yp,
                                           dinv, selfn, W0, b0)
            ytm = yp
        ph = _agg16(hs, pk, z16)
        h1, h1s, aggh = _stage_h1(ph, dinv, selfn, h, W1, b1)
        ph1 = _agg16(h1s, pk, z16)
        zv, zs = _stage_z(ph1, dinv, selfn, h1, aggx, aggh, W2, b2, W3)
        pz = _agg1(zs, pk, z1)
        yp, ys = _stage_y(pz, dinv, selfn, zv, ytm, b3)
        preds.append(yp)
        if t + 1 < t_fut:
            py = _agg1(ys, pk, z1)
    return jnp.stack(preds, axis=1)[:n]
